# baseline (speedup 1.0000x reference)
import functools

import jax
import jax.numpy as jnp
from jax import lax
from jax.experimental import pallas as pl
from jax.experimental.pallas import tpu as pltpu
from jax.experimental.pallas import tpu_sc as plsc

TRAIN_START = 900000
DIM = 32
LANES = 16

NC = 2
NS = 16
NW = NC * NS

CH = 512
GBLK = 128
NBLK = CH // GBLK


def _body(n_chunks, frozen_hbm, trainable_hbm, idx_hbm, out_hbm,
          idx_v, fidx_v, tidx_v, tbuf, obuf, semf, semt):
    c = lax.axis_index("c")
    s = lax.axis_index("s")
    wid = s * NC + c
    base = wid * (n_chunks * CH)

    def chunk_body(ci, carry):
        cbase = base + ci * CH
        pltpu.sync_copy(idx_hbm.at[pl.ds(cbase, CH)], idx_v)

        for g in range(CH // LANES):
            iv = idx_v[pl.ds(g * LANES, LANES)]
            is_t = iv >= TRAIN_START
            fidx_v[g // 8, pl.ds((g % 8) * LANES, LANES)] = jnp.where(is_t, 0, iv)
            tidx_v[g // 8, pl.ds((g % 8) * LANES, LANES)] = jnp.where(
                is_t, iv - TRAIN_START, 0)

        copies = []
        for j in range(NBLK):
            copies.append(pltpu.async_copy(
                frozen_hbm.at[fidx_v.at[j]], obuf.at[pl.ds(j * GBLK, GBLK)], semf))
            copies.append(pltpu.async_copy(
                trainable_hbm.at[tidx_v.at[j]], tbuf.at[pl.ds(j * GBLK, GBLK)], semt))
        for cp in copies:
            cp.wait()

        def group_body(q, rc):
            iv = idx_v[pl.ds(q * LANES, LANES)]
            rbase = q * LANES
            for k in range(LANES):
                @pl.when(iv[k] >= TRAIN_START)
                def _patch(k=k):
                    r = rbase + k
                    for h in range(DIM // LANES):
                        col = pl.ds(h * LANES, LANES)
                        obuf[r, col] = tbuf[r, col]
            return rc

        lax.fori_loop(0, CH // LANES, group_body, 0)
        pltpu.sync_copy(obuf, out_hbm.at[pl.ds(cbase, CH)])
        return carry

    lax.fori_loop(0, n_chunks, chunk_body, 0)


def kernel(frozen_weight, trainable_buffer, input_ids):
    b, s = input_ids.shape
    n = b * s
    assert n % (NW * CH) == 0
    n_chunks = n // (NW * CH)
    idx_flat = input_ids.reshape(n)

    k = pl.kernel(
        functools.partial(_body, n_chunks),
        out_type=jax.ShapeDtypeStruct((n, DIM), jnp.float32),
        mesh=plsc.VectorSubcoreMesh(core_axis_name="c", subcore_axis_name="s"),
        compiler_params=pltpu.CompilerParams(use_tc_tiling_on_sc=False),
        scratch_types=[
            pltpu.VMEM((CH,), jnp.int32),
            pltpu.VMEM((NBLK, GBLK), jnp.int32),
            pltpu.VMEM((NBLK, GBLK), jnp.int32),
            pltpu.VMEM((CH, DIM), jnp.float32),
            pltpu.VMEM((CH, DIM), jnp.float32),
            pltpu.SemaphoreType.DMA,
            pltpu.SemaphoreType.DMA,
        ],
    )
    out = k(frozen_weight, trainable_buffer, idx_flat)
    return out.reshape(b, s, DIM)


# speedup vs baseline: 7.1728x; 7.1728x over previous
import functools

import jax
import jax.numpy as jnp
from jax import lax
from jax.experimental import pallas as pl
from jax.experimental.pallas import tpu as pltpu
from jax.experimental.pallas import tpu_sc as plsc

TRAIN_START = 900000
DIM = 32
LANES = 16

NC = 2
NS = 16
NW = NC * NS

CH = 512
GBLK = 128
NBLK = CH // GBLK


def _body(n_chunks, frozen_hbm, trainable_hbm, idx_hbm, out_hbm,
          idx_v, fidx_v, tidx_v, tbuf, obuf, semf, semt):
    c = lax.axis_index("c")
    s = lax.axis_index("s")
    wid = s * NC + c
    base = wid * (n_chunks * CH)

    def chunk_body(ci, carry):
        cbase = base + ci * CH
        pltpu.sync_copy(idx_hbm.at[pl.ds(cbase, CH)], idx_v)

        for g in range(CH // LANES):
            iv = idx_v[pl.ds(g * LANES, LANES)]
            is_t = iv >= TRAIN_START
            dummy = lax.iota(jnp.int32, LANES) + (wid * CH + g * LANES)
            fidx_v[g // 8, pl.ds((g % 8) * LANES, LANES)] = jnp.where(is_t, dummy, iv)
            tidx_v[g // 8, pl.ds((g % 8) * LANES, LANES)] = jnp.where(
                is_t, iv - TRAIN_START, dummy)

        copies = []
        for j in range(NBLK):
            copies.append(pltpu.async_copy(
                frozen_hbm.at[fidx_v.at[j]], obuf.at[pl.ds(j * GBLK, GBLK)], semf))
            copies.append(pltpu.async_copy(
                trainable_hbm.at[tidx_v.at[j]], tbuf.at[pl.ds(j * GBLK, GBLK)], semt))
        for cp in copies:
            cp.wait()

        def group_body(q, rc):
            iv = idx_v[pl.ds(q * LANES, LANES)]
            rbase = q * LANES
            for k in range(LANES):
                @pl.when(iv[k] >= TRAIN_START)
                def _patch(k=k):
                    r = rbase + k
                    for h in range(DIM // LANES):
                        col = pl.ds(h * LANES, LANES)
                        obuf[r, col] = tbuf[r, col]
            return rc

        lax.fori_loop(0, CH // LANES, group_body, 0)
        pltpu.sync_copy(obuf, out_hbm.at[pl.ds(cbase, CH)])
        return carry

    lax.fori_loop(0, n_chunks, chunk_body, 0)


def kernel(frozen_weight, trainable_buffer, input_ids):
    b, s = input_ids.shape
    n = b * s
    assert n % (NW * CH) == 0
    n_chunks = n // (NW * CH)
    idx_flat = input_ids.reshape(n)

    k = pl.kernel(
        functools.partial(_body, n_chunks),
        out_type=jax.ShapeDtypeStruct((n, DIM), jnp.float32),
        mesh=plsc.VectorSubcoreMesh(core_axis_name="c", subcore_axis_name="s"),
        compiler_params=pltpu.CompilerParams(use_tc_tiling_on_sc=False),
        scratch_types=[
            pltpu.VMEM((CH,), jnp.int32),
            pltpu.VMEM((NBLK, GBLK), jnp.int32),
            pltpu.VMEM((NBLK, GBLK), jnp.int32),
            pltpu.VMEM((CH, DIM), jnp.float32),
            pltpu.VMEM((CH, DIM), jnp.float32),
            pltpu.SemaphoreType.DMA,
            pltpu.SemaphoreType.DMA,
        ],
    )
    out = k(frozen_weight, trainable_buffer, idx_flat)
    return out.reshape(b, s, DIM)
